# Initial kernel scaffold; baseline (speedup 1.0000x reference)
import jax
import jax.numpy as jnp
from jax.experimental import pallas as pl


def kernel(u_hyp, r_centres, angular_weight):
    raise NotImplementedError("write your pallas kernel here")


# speedup vs baseline: 1.6594x; 1.6594x over previous
import jax
import jax.numpy as jnp
from jax.experimental import pallas as pl
from jax.experimental.pallas import tpu as pltpu

_N_E = 8192
_E_DIM = 64
_RADIAL_BINS = 16
_ANGULAR_BINS = 512
_MAX_RADIUS = 18.0
_BETA = 0.25

_BLK = 2048


def _acosh(x):
    return jnp.log(x + jnp.sqrt(x - 1.0) * jnp.sqrt(x + 1.0))


def _vq_body(flat_ref, rc_ref, at_ref, a_ref,
             z_ref, loss_ref, perp_ref, usage_ref, emean_ref,
             counts_acc, loss_acc):
    pid = pl.program_id(0)
    nb = pl.num_programs(0)
    n_total = nb * _BLK

    @pl.when(pid == 0)
    def _init():
        counts_acc[...] = jnp.zeros_like(counts_acc)
        loss_acc[...] = jnp.zeros_like(loss_acc)

    flat = flat_ref[...]
    b = flat.shape[0]
    lane64 = jax.lax.broadcasted_iota(jnp.int32, (1, _E_DIM), 1)
    u_time = flat[:, 0:1]
    space = jnp.where(lane64 == 0, 0.0, flat)

    nrm = jnp.sqrt(jnp.sum(space * space, axis=1, keepdims=True))
    w = space / jnp.maximum(nrm, 1e-12)

    r = _acosh(jnp.maximum(u_time, 1.0 + 1e-7))
    rc_c = jnp.clip(rc_ref[...], 0.01, _MAX_RADIUS)
    lane128 = jax.lax.broadcasted_iota(jnp.int32, (1, 128), 1)
    dr = (r - rc_c) ** 2
    dr = jnp.where(lane128 < _RADIAL_BINS, dr, jnp.float32(jnp.inf))
    mr = jnp.min(dr, axis=1, keepdims=True)
    iota_b128 = jax.lax.broadcasted_iota(jnp.int32, (b, 128), 1)
    r_idx = jnp.min(jnp.where(dr == mr, iota_b128, jnp.int32(2**30)),
                    axis=1, keepdims=True)
    onehot_r = (iota_b128 == r_idx).astype(jnp.float32)
    r_hard = jnp.sum(onehot_r * rc_c, axis=1, keepdims=True)

    sim = jnp.dot(w, at_ref[...], preferred_element_type=jnp.float32)
    ms = jnp.max(sim, axis=1, keepdims=True)
    iota_b512 = jax.lax.broadcasted_iota(jnp.int32, (b, _ANGULAR_BINS), 1)
    w_idx = jnp.min(jnp.where(sim == ms, iota_b512, jnp.int32(2**30)),
                    axis=1, keepdims=True)
    onehot_w = (iota_b512 == w_idx).astype(jnp.float32)
    w_hard = jnp.dot(onehot_w, a_ref[...], preferred_element_type=jnp.float32)

    cnt = jax.lax.dot_general(onehot_r, onehot_w,
                              (((0,), (0,)), ((), ())),
                              preferred_element_type=jnp.float32)
    counts_acc[...] += cnt

    scale = jnp.tanh(r_hard * 0.5)
    xq = scale * w_hard
    x2 = jnp.sum(xq * xq, axis=1, keepdims=True)
    denom = jnp.maximum(1.0 - x2, 1e-7)
    xsp = (2.0 / denom) * xq
    t2 = jnp.sqrt(1.0 + jnp.sum(xsp * xsp, axis=1, keepdims=True))

    inner = jnp.sum(flat * xsp, axis=1, keepdims=True) - u_time * t2
    dist = _acosh(jnp.maximum(-inner, 1.0 + 1e-7))
    loss_acc[...] += jnp.sum(dist).reshape(1, 1)

    xq_full = jnp.where(lane64 == 0, t2, xsp)
    zf = flat + (xq_full - flat)
    zsp = jnp.where(lane64 == 0, 0.0, zf)
    zt = jnp.sqrt(1.0 + jnp.sum(zsp * zsp, axis=1, keepdims=True))
    z_ref[...] = jnp.where(lane64 == 0, zt, zf)

    @pl.when(pid == nb - 1)
    def _fin():
        e_mean = counts_acc[...] / jnp.float32(n_total)
        emean_ref[...] = e_mean[:_RADIAL_BINS, :]
        ent = jnp.sum(e_mean * jnp.log(e_mean + 1e-10))
        perp_ref[...] = jnp.exp(-ent).reshape(1, 1)
        usage_ref[...] = (jnp.sum((e_mean > 0).astype(jnp.float32))
                          / jnp.float32(_N_E)).reshape(1, 1)
        loss_ref[...] = _BETA * loss_acc[...] / jnp.float32(n_total)


def kernel(u_hyp, r_centres, angular_weight):
    u_shape = u_hyp.shape
    flat = u_hyp.reshape(-1, _E_DIM)
    n = flat.shape[0]
    grid = n // _BLK

    a64 = jnp.concatenate(
        [jnp.zeros((_ANGULAR_BINS, 1), angular_weight.dtype), angular_weight],
        axis=1)
    at64 = a64.T
    rc_pad = jnp.zeros((1, 128), jnp.float32).at[0, :_RADIAL_BINS].set(r_centres)

    z, loss, perp, usage, emean = pl.pallas_call(
        _vq_body,
        grid=(grid,),
        in_specs=[
            pl.BlockSpec((_BLK, _E_DIM), lambda i: (i, 0)),
            pl.BlockSpec((1, 128), lambda i: (0, 0)),
            pl.BlockSpec((_E_DIM, _ANGULAR_BINS), lambda i: (0, 0)),
            pl.BlockSpec((_ANGULAR_BINS, _E_DIM), lambda i: (0, 0)),
        ],
        out_specs=[
            pl.BlockSpec((_BLK, _E_DIM), lambda i: (i, 0)),
            pl.BlockSpec((1, 1), lambda i: (0, 0)),
            pl.BlockSpec((1, 1), lambda i: (0, 0)),
            pl.BlockSpec((1, 1), lambda i: (0, 0)),
            pl.BlockSpec((_RADIAL_BINS, _ANGULAR_BINS), lambda i: (0, 0)),
        ],
        out_shape=[
            jax.ShapeDtypeStruct((n, _E_DIM), jnp.float32),
            jax.ShapeDtypeStruct((1, 1), jnp.float32),
            jax.ShapeDtypeStruct((1, 1), jnp.float32),
            jax.ShapeDtypeStruct((1, 1), jnp.float32),
            jax.ShapeDtypeStruct((_RADIAL_BINS, _ANGULAR_BINS), jnp.float32),
        ],
        scratch_shapes=[
            pltpu.VMEM((128, _ANGULAR_BINS), jnp.float32),
            pltpu.VMEM((1, 1), jnp.float32),
        ],
    )(flat, rc_pad, at64, a64)

    return (loss[0, 0], z.reshape(u_shape), perp[0, 0], usage[0, 0],
            emean.reshape(-1))


# speedup vs baseline: 3.2975x; 1.9872x over previous
import jax
import jax.numpy as jnp
from jax.experimental import pallas as pl
from jax.experimental.pallas import tpu as pltpu

_N_E = 8192
_E_DIM = 64
_RADIAL_BINS = 16
_ANGULAR_BINS = 512
_MAX_RADIUS = 18.0
_BETA = 0.25

_BLK = 2048


def _acosh(x):
    return jnp.log(x + jnp.sqrt(x - 1.0) * jnp.sqrt(x + 1.0))


def _vq_body(flatt_ref, rc_ref, a_ref,
             z_ref, loss_ref, perp_ref, usage_ref, emean_ref,
             counts_acc, loss_acc):
    pid = pl.program_id(0)
    nb = pl.num_programs(0)
    n_total = nb * _BLK

    @pl.when(pid == 0)
    def _init():
        counts_acc[...] = jnp.zeros_like(counts_acc)
        loss_acc[...] = jnp.zeros_like(loss_acc)

    flatb = flatt_ref[...]
    u_time = flatb[0:1, :]
    row = jax.lax.broadcasted_iota(jnp.int32, (_E_DIM, 1), 0)
    space = jnp.where(row == 0, 0.0, flatb)

    nrm = jnp.sqrt(jnp.sum(space * space, axis=0, keepdims=True))
    w = space / jnp.maximum(nrm, 1e-12)

    sim = jax.lax.dot_general(a_ref[...], w,
                              (((1,), (0,)), ((), ())),
                              preferred_element_type=jnp.float32)
    ms = jnp.max(sim, axis=0, keepdims=True)
    onehot_w = (sim == ms).astype(jnp.float32)
    w_hard = jax.lax.dot_general(a_ref[...], onehot_w,
                                 (((0,), (0,)), ((), ())),
                                 preferred_element_type=jnp.float32)

    r = _acosh(jnp.maximum(u_time, 1.0 + 1e-7))
    rc_c = jnp.clip(rc_ref[...][:, 0:1], 0.01, _MAX_RADIUS)
    dr = (r - rc_c) ** 2
    mr = jnp.min(dr, axis=0, keepdims=True)
    onehot_r = (dr == mr).astype(jnp.float32)
    r_hard = jnp.sum(onehot_r * rc_c, axis=0, keepdims=True)

    cnt = jax.lax.dot_general(onehot_r, onehot_w,
                              (((1,), (1,)), ((), ())),
                              preferred_element_type=jnp.float32)
    counts_acc[...] += cnt

    scale = jnp.tanh(r_hard * 0.5)
    xq = scale * w_hard
    x2 = jnp.sum(xq * xq, axis=0, keepdims=True)
    denom = jnp.maximum(1.0 - x2, 1e-7)
    xsp = (2.0 / denom) * xq
    t2 = jnp.sqrt(1.0 + jnp.sum(xsp * xsp, axis=0, keepdims=True))

    inner = jnp.sum(flatb * xsp, axis=0, keepdims=True) - u_time * t2
    dist = _acosh(jnp.maximum(-inner, 1.0 + 1e-7))
    loss_acc[...] += jnp.sum(dist).reshape(1, 1)

    xq_full = jnp.where(row == 0, t2, xsp)
    zf = flatb + (xq_full - flatb)
    zsp = jnp.where(row == 0, 0.0, zf)
    zt = jnp.sqrt(1.0 + jnp.sum(zsp * zsp, axis=0, keepdims=True))
    z_ref[...] = jnp.where(row == 0, zt, zf)

    @pl.when(pid == nb - 1)
    def _fin():
        e_mean = counts_acc[...] / jnp.float32(n_total)
        emean_ref[...] = e_mean
        ent = jnp.sum(e_mean * jnp.log(e_mean + 1e-10))
        perp_ref[...] = jnp.exp(-ent).reshape(1, 1)
        usage_ref[...] = (jnp.sum((e_mean > 0).astype(jnp.float32))
                          / jnp.float32(_N_E)).reshape(1, 1)
        loss_ref[...] = _BETA * loss_acc[...] / jnp.float32(n_total)


def kernel(u_hyp, r_centres, angular_weight):
    u_shape = u_hyp.shape
    flat = u_hyp.reshape(-1, _E_DIM)
    n = flat.shape[0]
    grid = n // _BLK

    flatt = flat.T
    a64 = jnp.concatenate(
        [jnp.zeros((_ANGULAR_BINS, 1), angular_weight.dtype), angular_weight],
        axis=1)
    rc_rep = jnp.broadcast_to(r_centres.reshape(_RADIAL_BINS, 1),
                              (_RADIAL_BINS, 128))

    zt, loss, perp, usage, emean = pl.pallas_call(
        _vq_body,
        grid=(grid,),
        in_specs=[
            pl.BlockSpec((_E_DIM, _BLK), lambda i: (0, i)),
            pl.BlockSpec((_RADIAL_BINS, 128), lambda i: (0, 0)),
            pl.BlockSpec((_ANGULAR_BINS, _E_DIM), lambda i: (0, 0)),
        ],
        out_specs=[
            pl.BlockSpec((_E_DIM, _BLK), lambda i: (0, i)),
            pl.BlockSpec((1, 1), lambda i: (0, 0)),
            pl.BlockSpec((1, 1), lambda i: (0, 0)),
            pl.BlockSpec((1, 1), lambda i: (0, 0)),
            pl.BlockSpec((_RADIAL_BINS, _ANGULAR_BINS), lambda i: (0, 0)),
        ],
        out_shape=[
            jax.ShapeDtypeStruct((_E_DIM, n), jnp.float32),
            jax.ShapeDtypeStruct((1, 1), jnp.float32),
            jax.ShapeDtypeStruct((1, 1), jnp.float32),
            jax.ShapeDtypeStruct((1, 1), jnp.float32),
            jax.ShapeDtypeStruct((_RADIAL_BINS, _ANGULAR_BINS), jnp.float32),
        ],
        scratch_shapes=[
            pltpu.VMEM((_RADIAL_BINS, _ANGULAR_BINS), jnp.float32),
            pltpu.VMEM((1, 1), jnp.float32),
        ],
    )(flatt, rc_rep, a64)

    return (loss[0, 0], zt.T.reshape(u_shape), perp[0, 0], usage[0, 0],
            emean.reshape(-1))
